# baseline (speedup 1.0000x reference)
import jax
import jax.numpy as jnp
from jax.experimental import pallas as pl
from jax.experimental.pallas import tpu as pltpu

_BN = 512


def _vq_argmin_kernel(x_ref, c_ref, o_ref):
    x = x_ref[...]
    c = c_ref[...]
    k = c.shape[0]
    xc = jax.lax.dot_general(
        x, c, (((1,), (1,)), ((), ())), preferred_element_type=jnp.float32
    )
    x2 = jnp.sum(x * x, axis=1, keepdims=True)
    c2 = jnp.sum(c * c, axis=1)[None, :]
    d2 = x2 + c2 - 2.0 * xc
    dist = jnp.sqrt(jnp.maximum(d2, 0.0))
    m = jnp.min(dist, axis=1, keepdims=True)
    ids = jax.lax.broadcasted_iota(jnp.int32, dist.shape, 1)
    o_ref[...] = jnp.min(jnp.where(dist == m, ids, jnp.int32(k)), axis=1)


def kernel(x, centroids):
    n, d = x.shape
    k, _ = centroids.shape
    grid = (n // _BN,)
    return pl.pallas_call(
        _vq_argmin_kernel,
        grid=grid,
        in_specs=[
            pl.BlockSpec((_BN, d), lambda i: (i, 0)),
            pl.BlockSpec((k, d), lambda i: (0, 0)),
        ],
        out_specs=pl.BlockSpec((_BN,), lambda i: (i,)),
        out_shape=jax.ShapeDtypeStruct((n,), jnp.int32),
        compiler_params=pltpu.CompilerParams(
            dimension_semantics=("parallel",),
        ),
    )(x, centroids)


# speedup vs baseline: 1.0235x; 1.0235x over previous
import jax
import jax.numpy as jnp
from jax.experimental import pallas as pl
from jax.experimental.pallas import tpu as pltpu

_BN = 512


_CH = 256


def _vq_argmin_kernel(x_ref, c_ref, o_ref):
    x = x_ref[...]
    k = c_ref.shape[0]
    x2 = jnp.sum(x * x, axis=1, keepdims=True)
    run_v = run_i = None
    for j in range(k // _CH):
        cj = c_ref[pl.ds(j * _CH, _CH), :]
        xc = jax.lax.dot_general(
            x, cj, (((1,), (1,)), ((), ())), preferred_element_type=jnp.float32
        )
        c2 = jnp.sum(cj * cj, axis=1)[None, :]
        d2 = x2 + c2 - 2.0 * xc
        dist = jnp.sqrt(jnp.maximum(d2, 0.0))
        ids = jax.lax.broadcasted_iota(jnp.int32, dist.shape, 1) + j * _CH
        if run_v is None:
            run_v, run_i = dist, ids
        else:
            lt = dist < run_v
            run_v = jnp.where(lt, dist, run_v)
            run_i = jnp.where(lt, ids, run_i)
    m = jnp.min(run_v, axis=1, keepdims=True)
    cand = jnp.where(run_v == m, run_i, jnp.int32(k))
    o_ref[...] = jnp.min(cand, axis=1)


def kernel(x, centroids):
    n, d = x.shape
    k, _ = centroids.shape
    grid = (n // _BN,)
    return pl.pallas_call(
        _vq_argmin_kernel,
        grid=grid,
        in_specs=[
            pl.BlockSpec((_BN, d), lambda i: (i, 0)),
            pl.BlockSpec((k, d), lambda i: (0, 0)),
        ],
        out_specs=pl.BlockSpec((_BN,), lambda i: (i,)),
        out_shape=jax.ShapeDtypeStruct((n,), jnp.int32),
        compiler_params=pltpu.CompilerParams(
            dimension_semantics=("parallel",),
        ),
    )(x, centroids)
